# baseline (speedup 1.0000x reference)
import jax
import jax.numpy as jnp
from jax import lax
from jax.experimental import pallas as pl
from jax.experimental.pallas import tpu as pltpu
from jax.experimental.pallas import tpu_sc as plsc

_N = 32768
_B = 16
_D = 256
_NC = 2
_NS = 16
_NW = _NC * _NS
_L = 16
_C = 128
_DV = _D // _L


def _lane_select(vec, s):
    lane = lax.iota(jnp.int32, _L)
    return jnp.max(jnp.where(lane == s, vec, 0))


def _sum_body(x_hbm, len_hbm, part_hbm, len_v, buf0, buf1, acc, sem0, sem1):
    cid = lax.axis_index("c")
    sid = lax.axis_index("s")
    wid = sid * _NC + cid

    pltpu.sync_copy(len_hbm, len_v)
    lens = len_v[...]
    csum = plsc.cumsum(lens)
    total = jnp.max(csum)

    zero = jnp.zeros((_L,), jnp.float32)

    for s in range(_B):
        for j in range(_DV):
            acc[s, pl.ds(j * _L, _L)] = zero

    offs = [jnp.int32(0)] + [_lane_select(csum, s) for s in range(_B)]

    nchunks = (total + _C - 1) // _C
    kw = (nchunks - wid + _NW - 1) // _NW

    bufs = (buf0, buf1)
    sems = (sem0, sem1)

    def copy_of(k, slot):
        row0 = (wid + k * _NW) * _C
        return pltpu.make_async_copy(
            x_hbm.at[pl.ds(row0, _C), :], bufs[slot], sems[slot]
        )

    @pl.when(kw > 0)
    def _():
        copy_of(0, 0).start()

    def process(k, slot):
        buf = bufs[slot]
        row0 = (wid + k * _NW) * _C

        @pl.when(k + 1 < kw)
        def _():
            copy_of(k + 1, 1 - slot).start()

        copy_of(k, slot).wait()
        row1 = jnp.minimum(row0 + _C, total)
        for s in range(_B):
            a = jnp.maximum(row0, offs[s])
            b = jnp.minimum(row1, offs[s + 1])

            @pl.when(b > a)
            def _():
                la = a - row0
                lb = b - row0
                n4 = (lb - la) & ~3

                def rbody4(i, carry):
                    rr = la + i * 4
                    c = carry
                    for u in range(4):
                        c = tuple(
                            c[j] + buf[rr + u, pl.ds(j * _L, _L)]
                            for j in range(_DV)
                        )
                    return c

                run = lax.fori_loop(0, n4 // 4, rbody4, (zero,) * _DV)

                def rbody1(rr, carry):
                    return tuple(
                        carry[j] + buf[rr, pl.ds(j * _L, _L)]
                        for j in range(_DV)
                    )

                run = lax.fori_loop(la + n4, lb, rbody1, run)
                for j in range(_DV):
                    o = j * _L
                    acc[s, pl.ds(o, _L)] = acc[s, pl.ds(o, _L)] + run[j]

    def pair_body(i, c):
        k = i * 2
        for slot in range(2):
            @pl.when(k + slot < kw)
            def _():
                process(k + slot, slot)
        return c

    lax.fori_loop(0, (kw + 1) // 2, pair_body, 0)
    pltpu.sync_copy(acc, part_hbm.at[pl.ds(wid * _B, _B), :])


def _combine_body(part_hbm, len_hbm, out_hbm, idx_v, rows_v, len_v, out_v, sem):
    cid = lax.axis_index("c")
    sid = lax.axis_index("s")
    wid = sid * _NC + cid

    @pl.when(wid < _B)
    def _():
        lane = lax.iota(jnp.int32, _L)
        idx_v[pl.ds(0, _L)] = lane * _B + wid
        idx_v[pl.ds(_L, _L)] = (lane + _L) * _B + wid
        pltpu.async_copy(part_hbm.at[idx_v], rows_v, sem).wait()

        pltpu.sync_copy(len_hbm, len_v)
        lens = len_v[...]
        cnt = jnp.max(jnp.where(lane == wid, jnp.maximum(lens, 1), 0))
        cnt_vec = jnp.full((_L,), cnt, jnp.int32).astype(jnp.float32)
        recip = jnp.ones((_L,), jnp.float32) / cnt_vec

        def rbody(r, carry):
            return tuple(
                carry[j] + rows_v[r, pl.ds(j * _L, _L)] for j in range(_DV)
            )

        tot = lax.fori_loop(
            0, _NW, rbody, (jnp.zeros((_L,), jnp.float32),) * _DV
        )
        for j in range(_DV):
            out_v[0, pl.ds(j * _L, _L)] = tot[j] * recip
        pltpu.sync_copy(out_v, out_hbm.at[pl.ds(wid, 1), :])


_mesh = plsc.VectorSubcoreMesh(core_axis_name="c", subcore_axis_name="s")
_params = pltpu.CompilerParams(needs_layout_passes=False)

_sum_call = pl.kernel(
    _sum_body,
    out_type=jax.ShapeDtypeStruct((_NW * _B, _D), jnp.float32),
    mesh=_mesh,
    compiler_params=_params,
    scratch_types=[
        pltpu.VMEM((_L,), jnp.int32),
        pltpu.VMEM((_C, _D), jnp.float32),
        pltpu.VMEM((_C, _D), jnp.float32),
        pltpu.VMEM((_B, _D), jnp.float32),
        pltpu.SemaphoreType.DMA,
        pltpu.SemaphoreType.DMA,
    ],
)

_combine_call = pl.kernel(
    _combine_body,
    out_type=jax.ShapeDtypeStruct((_B, _D), jnp.float32),
    mesh=_mesh,
    compiler_params=_params,
    scratch_types=[
        pltpu.VMEM((_NW,), jnp.int32),
        pltpu.VMEM((_NW, _D), jnp.float32),
        pltpu.VMEM((_L,), jnp.int32),
        pltpu.VMEM((1, _D), jnp.float32),
        pltpu.SemaphoreType.DMA,
    ],
)


def kernel(x, batch_lengths):
    part = _sum_call(x, batch_lengths)
    return _combine_call(part, batch_lengths)


# speedup vs baseline: 1.3502x; 1.3502x over previous
import jax
import jax.numpy as jnp
from jax import lax
from jax.experimental import pallas as pl
from jax.experimental.pallas import tpu as pltpu
from jax.experimental.pallas import tpu_sc as plsc

_N = 32768
_B = 16
_D = 256
_NC = 2
_NS = 16
_NW = _NC * _NS
_L = 16
_C = 128
_DV = _D // _L


def _lane_select(vec, s):
    lane = lax.iota(jnp.int32, _L)
    return jnp.max(jnp.where(lane == s, vec, 0))


def _sum_body(x_hbm, len_hbm, part_hbm, len_v, buf0, buf1, acc, sem0, sem1):
    cid = lax.axis_index("c")
    sid = lax.axis_index("s")
    wid = sid * _NC + cid

    pltpu.sync_copy(len_hbm, len_v)
    lens = len_v[...]
    csum = plsc.cumsum(lens)
    total = jnp.max(csum)

    zero = jnp.zeros((_L,), jnp.float32)

    for s in range(_B):
        for j in range(_DV):
            acc[s, pl.ds(j * _L, _L)] = zero

    offs = [jnp.int32(0)] + [_lane_select(csum, s) for s in range(_B)]

    nchunks = (total + _C - 1) // _C
    kw = (nchunks - wid + _NW - 1) // _NW

    bufs = (buf0, buf1)
    sems = (sem0, sem1)

    def copy_of(k, slot):
        row0 = (wid + k * _NW) * _C
        return pltpu.make_async_copy(
            x_hbm.at[pl.ds(row0, _C), :], bufs[slot], sems[slot]
        )

    @pl.when(kw > 0)
    def _():
        copy_of(0, 0).start()

    def process(k, slot):
        buf = bufs[slot]
        row0 = (wid + k * _NW) * _C

        @pl.when(k + 1 < kw)
        def _():
            copy_of(k + 1, 1 - slot).start()

        copy_of(k, slot).wait()
        row1 = jnp.minimum(row0 + _C, total)
        for s in range(_B):
            a = jnp.maximum(row0, offs[s])
            b = jnp.minimum(row1, offs[s + 1])

            @pl.when(b > a)
            def _():
                def rbody(rr, carry):
                    return tuple(
                        carry[j] + buf[rr, pl.ds(j * _L, _L)]
                        for j in range(_DV)
                    )

                run = lax.fori_loop(a - row0, b - row0, rbody, (zero,) * _DV)
                for j in range(_DV):
                    o = j * _L
                    acc[s, pl.ds(o, _L)] = acc[s, pl.ds(o, _L)] + run[j]

    def pair_body(i, c):
        k = i * 2
        for slot in range(2):
            @pl.when(k + slot < kw)
            def _():
                process(k + slot, slot)
        return c

    lax.fori_loop(0, (kw + 1) // 2, pair_body, 0)
    pltpu.sync_copy(acc, part_hbm.at[pl.ds(wid * _B, _B), :])


def _combine_body(part_hbm, len_hbm, out_hbm, idx_v, rows_v, len_v, out_v, sem):
    cid = lax.axis_index("c")
    sid = lax.axis_index("s")
    wid = sid * _NC + cid

    @pl.when(wid < _B)
    def _():
        lane = lax.iota(jnp.int32, _L)
        idx_v[pl.ds(0, _L)] = lane * _B + wid
        idx_v[pl.ds(_L, _L)] = (lane + _L) * _B + wid
        pltpu.async_copy(part_hbm.at[idx_v], rows_v, sem).wait()

        pltpu.sync_copy(len_hbm, len_v)
        lens = len_v[...]
        cnt = jnp.max(jnp.where(lane == wid, jnp.maximum(lens, 1), 0))
        cnt_vec = jnp.full((_L,), cnt, jnp.int32).astype(jnp.float32)
        recip = jnp.ones((_L,), jnp.float32) / cnt_vec

        def rbody(r, carry):
            return tuple(
                carry[j] + rows_v[r, pl.ds(j * _L, _L)] for j in range(_DV)
            )

        tot = lax.fori_loop(
            0, _NW, rbody, (jnp.zeros((_L,), jnp.float32),) * _DV
        )
        for j in range(_DV):
            out_v[0, pl.ds(j * _L, _L)] = tot[j] * recip
        pltpu.sync_copy(out_v, out_hbm.at[pl.ds(wid, 1), :])


_mesh = plsc.VectorSubcoreMesh(core_axis_name="c", subcore_axis_name="s")
_params = pltpu.CompilerParams(needs_layout_passes=False)

_sum_call = pl.kernel(
    _sum_body,
    out_type=jax.ShapeDtypeStruct((_NW * _B, _D), jnp.float32),
    mesh=_mesh,
    compiler_params=_params,
    scratch_types=[
        pltpu.VMEM((_L,), jnp.int32),
        pltpu.VMEM((_C, _D), jnp.float32),
        pltpu.VMEM((_C, _D), jnp.float32),
        pltpu.VMEM((_B, _D), jnp.float32),
        pltpu.SemaphoreType.DMA,
        pltpu.SemaphoreType.DMA,
    ],
)

_combine_call = pl.kernel(
    _combine_body,
    out_type=jax.ShapeDtypeStruct((_B, _D), jnp.float32),
    mesh=_mesh,
    compiler_params=_params,
    scratch_types=[
        pltpu.VMEM((_NW,), jnp.int32),
        pltpu.VMEM((_NW, _D), jnp.float32),
        pltpu.VMEM((_L,), jnp.int32),
        pltpu.VMEM((1, _D), jnp.float32),
        pltpu.SemaphoreType.DMA,
    ],
)


def _tc_combine_body(part_ref, len_ref, out_ref):
    s = part_ref[0:_B, :]
    for w in range(1, _NW):
        s = s + part_ref[w * _B:(w + 1) * _B, :]
    cnt = jnp.maximum(len_ref[...], 1).astype(jnp.float32)
    out_ref[...] = s / cnt[:, None]


_tc_combine = pl.pallas_call(
    _tc_combine_body,
    out_shape=jax.ShapeDtypeStruct((_B, _D), jnp.float32),
)


def kernel(x, batch_lengths):
    part = _sum_call(x, batch_lengths)
    return _tc_combine(part, batch_lengths)


# speedup vs baseline: 1.6858x; 1.2485x over previous
import jax
import jax.numpy as jnp
from jax import lax
from jax.experimental import pallas as pl
from jax.experimental.pallas import tpu as pltpu
from jax.experimental.pallas import tpu_sc as plsc

_N = 32768
_B = 16
_D = 256
_NC = 2
_NS = 16
_NW = _NC * _NS
_L = 16
_C = 128
_DV = _D // _L


def _sum_body(x_hbm, len_hbm, part_hbm, len_v, buf0, buf1, acc, sem0, sem1):
    cid = lax.axis_index("c")
    sid = lax.axis_index("s")
    wid = sid * _NC + cid

    pltpu.sync_copy(len_hbm, len_v)
    lens = len_v[...]
    csum = plsc.cumsum(lens)
    total = jnp.max(csum)
    lane = lax.iota(jnp.int32, _L)

    zero = jnp.zeros((_L,), jnp.float32)

    def zbody(i, c):
        for j in range(_DV):
            acc[i, pl.ds(j * _L, _L)] = zero
        return c

    lax.fori_loop(0, _B, zbody, 0)

    nchunks = (total + _C - 1) // _C
    kw = (nchunks - wid + _NW - 1) // _NW

    bufs = (buf0, buf1)
    sems = (sem0, sem1)

    def copy_of(k, slot):
        row0 = (wid + k * _NW) * _C
        return pltpu.make_async_copy(
            x_hbm.at[pl.ds(row0, _C), :], bufs[slot], sems[slot]
        )

    @pl.when(kw > 0)
    def _():
        copy_of(0, 0).start()

    def process(k, slot):
        buf = bufs[slot]
        row0 = (wid + k * _NW) * _C

        @pl.when(k + 1 < kw)
        def _():
            copy_of(k + 1, 1 - slot).start()

        copy_of(k, slot).wait()
        row1 = jnp.minimum(row0 + _C, total)
        s0 = jnp.sum(jnp.where(csum <= row0, 1, 0))

        def run_cond(st):
            return st[1] < row1

        def run_body(st):
            s, a = st
            end_s = jnp.max(jnp.where(lane == s, csum, 0))
            b = jnp.minimum(end_s, row1)

            def rbody(rr, carry):
                return tuple(
                    carry[j] + buf[rr, pl.ds(j * _L, _L)]
                    for j in range(_DV)
                )

            run = lax.fori_loop(a - row0, b - row0, rbody, (zero,) * _DV)
            for j in range(_DV):
                o = j * _L
                acc[s, pl.ds(o, _L)] = acc[s, pl.ds(o, _L)] + run[j]
            return (s + 1, b)

        lax.while_loop(run_cond, run_body, (s0, row0))

    def pair_body(i, c):
        k = i * 2
        for slot in range(2):
            @pl.when(k + slot < kw)
            def _():
                process(k + slot, slot)
        return c

    lax.fori_loop(0, (kw + 1) // 2, pair_body, 0)
    pltpu.sync_copy(acc, part_hbm.at[pl.ds(wid * _B, _B), :])


_mesh = plsc.VectorSubcoreMesh(core_axis_name="c", subcore_axis_name="s")
_params = pltpu.CompilerParams(needs_layout_passes=False)

_sum_call = pl.kernel(
    _sum_body,
    out_type=jax.ShapeDtypeStruct((_NW * _B, _D), jnp.float32),
    mesh=_mesh,
    compiler_params=_params,
    scratch_types=[
        pltpu.VMEM((_L,), jnp.int32),
        pltpu.VMEM((_C, _D), jnp.float32),
        pltpu.VMEM((_C, _D), jnp.float32),
        pltpu.VMEM((_B, _D), jnp.float32),
        pltpu.SemaphoreType.DMA,
        pltpu.SemaphoreType.DMA,
    ],
)


def _tc_combine_body(part_ref, len_ref, out_ref):
    s = part_ref[0:_B, :]
    for w in range(1, _NW):
        s = s + part_ref[w * _B:(w + 1) * _B, :]
    cnt = jnp.maximum(len_ref[...], 1).astype(jnp.float32)
    out_ref[...] = s / cnt[:, None]


_tc_combine = pl.pallas_call(
    _tc_combine_body,
    out_shape=jax.ShapeDtypeStruct((_B, _D), jnp.float32),
)


def kernel(x, batch_lengths):
    part = _sum_call(x, batch_lengths)
    return _tc_combine(part, batch_lengths)


# speedup vs baseline: 1.7127x; 1.0159x over previous
import jax
import jax.numpy as jnp
from jax import lax
from jax.experimental import pallas as pl
from jax.experimental.pallas import tpu as pltpu
from jax.experimental.pallas import tpu_sc as plsc

_N = 32768
_B = 16
_D = 256
_NC = 2
_NS = 16
_L = 16
_C = 256
_H = _D // _NC
_HV = _H // _L


def _sum_body(x_hbm, len_hbm, out_hbm, len_v, buf0, buf1, acc, rows16,
              shared, sem0, sem1, psem):
    cid = lax.axis_index("c")
    sid = lax.axis_index("s")
    col0 = cid * _H

    pltpu.sync_copy(len_hbm, len_v)
    lens = len_v[...]
    csum = plsc.cumsum(lens)
    total = jnp.max(csum)
    lane = lax.iota(jnp.int32, _L)

    zero = jnp.zeros((_L,), jnp.float32)

    def zbody(i, c):
        for j in range(_HV):
            acc[i, pl.ds(j * _L, _L)] = zero
        return c

    lax.fori_loop(0, _B, zbody, 0)

    nchunks = (total + _C - 1) // _C
    kw = (nchunks - sid + _NS - 1) // _NS

    bufs = (buf0, buf1)
    sems = (sem0, sem1)

    def copy_of(k, slot):
        row0 = (sid + k * _NS) * _C
        return pltpu.make_async_copy(
            x_hbm.at[pl.ds(row0, _C), pl.ds(col0, _H)], bufs[slot],
            sems[slot]
        )

    @pl.when(kw > 0)
    def _():
        copy_of(0, 0).start()

    def process(k, slot):
        buf = bufs[slot]
        row0 = (sid + k * _NS) * _C

        @pl.when(k + 1 < kw)
        def _():
            copy_of(k + 1, 1 - slot).start()

        copy_of(k, slot).wait()
        row1 = jnp.minimum(row0 + _C, total)
        s0 = jnp.sum(jnp.where(csum <= row0, 1, 0))

        def run_cond(st):
            return st[1] < row1

        def run_body(st):
            s, a = st
            end_s = jnp.max(jnp.where(lane == s, csum, 0))
            b = jnp.minimum(end_s, row1)

            def rbody(rr, carry):
                return tuple(
                    carry[j] + buf[rr, pl.ds(j * _L, _L)]
                    for j in range(_HV)
                )

            run = lax.fori_loop(a - row0, b - row0, rbody, (zero,) * _HV)
            for j in range(_HV):
                o = j * _L
                acc[s, pl.ds(o, _L)] = acc[s, pl.ds(o, _L)] + run[j]
            return (s + 1, b)

        lax.while_loop(run_cond, run_body, (s0, row0))

    def pair_body(i, c):
        k = i * 2
        for slot in range(2):
            @pl.when(k + slot < kw)
            def _():
                process(k + slot, slot)
        return c

    lax.fori_loop(0, (kw + 1) // 2, pair_body, 0)

    for s in range(_B):
        pltpu.sync_copy(acc.at[s], shared.at[s, sid])
    plsc.subcore_barrier()
    plsc.subcore_barrier()

    pltpu.sync_copy(shared.at[sid], rows16)

    def fbody(w, carry):
        return tuple(
            carry[j] + rows16[w, pl.ds(j * _L, _L)] for j in range(_HV)
        )

    tot = lax.fori_loop(0, _NS, fbody, (zero,) * _HV)
    cnt = jnp.max(jnp.where(lane == sid, jnp.maximum(lens, 1), 0))
    cnt_vec = jnp.full((_L,), cnt, jnp.int32).astype(jnp.float32)
    recip = jnp.ones((_L,), jnp.float32) / cnt_vec
    for j in range(_HV):
        rows16[0, pl.ds(j * _L, _L)] = tot[j] * recip
    pltpu.sync_copy(rows16.at[0], out_hbm.at[sid, pl.ds(col0, _H)])


_mesh = plsc.VectorSubcoreMesh(core_axis_name="c", subcore_axis_name="s")
_params = pltpu.CompilerParams(needs_layout_passes=False)

_sum_call = pl.kernel(
    _sum_body,
    out_type=jax.ShapeDtypeStruct((_B, _D), jnp.float32),
    mesh=_mesh,
    compiler_params=_params,
    scratch_types=[
        pltpu.VMEM((_L,), jnp.int32),
        pltpu.VMEM((_C, _H), jnp.float32),
        pltpu.VMEM((_C, _H), jnp.float32),
        pltpu.VMEM((_B, _H), jnp.float32),
        pltpu.VMEM((_NS, _H), jnp.float32),
        pltpu.VMEM_SHARED((_B, _NS, _H), jnp.float32),
        pltpu.SemaphoreType.DMA,
        pltpu.SemaphoreType.DMA,
        pltpu.SemaphoreType.DMA,
    ],
)


def kernel(x, batch_lengths):
    return _sum_call(x, batch_lengths)
